# baseline (speedup 1.0000x reference)
import functools

import jax
import jax.numpy as jnp
from jax import lax
from jax.experimental import pallas as pl
from jax.experimental.pallas import tpu as pltpu
from jax.experimental.pallas import tpu_sc as plsc

_P1, _P2 = 778, 3000
_BPP = 2


def _split3_bf16(t):
    hi = t.astype(jnp.bfloat16)
    r1 = t - hi.astype(jnp.float32)
    mid = r1.astype(jnp.bfloat16)
    lo = (r1 - mid.astype(jnp.float32)).astype(jnp.bfloat16)
    return hi, mid, lo


def _nn_body(x_ref, y_ref, n_ref, y2xu_ref, x2y_ref, yidx_ref):
  del n_ref
  for s in range(_BPP):
    xb = x_ref[s]
    yt = y_ref[s]

    d = None
    for c in range(3):
        diff = yt[c : c + 1, :] - xb[:, c : c + 1]
        sq = diff * diff
        d = sq if d is None else d + sq

    row_min = jnp.min(d, axis=1, keepdims=True)
    x2y_ref[s] = jnp.sqrt(row_min)

    col_min = jnp.min(d, axis=0, keepdims=True)
    onehot = jnp.where(d == col_min, 1.0, 0.0).astype(jnp.bfloat16)

    ii = jax.lax.broadcasted_iota(jnp.int32, (_P1, 1), 0).astype(jnp.float32)
    ones = jnp.ones((_P1, 1), jnp.float32)
    table = jnp.concatenate([ii, ones, ii * ii], axis=1)
    dims = (((0,), (0,)), ((), ()))
    gath = None
    for limb in _split3_bf16(table):
        part = jax.lax.dot_general(
            limb, onehot, dims, preferred_element_type=jnp.float32
        )
        gath = part if gath is None else gath + part

    ssum = gath[0:1]
    cnt = gath[1:2]
    q = gath[2:3]
    tie_lo = 0.5 * (ssum - jnp.sqrt(jnp.maximum(2.0 * q - ssum * ssum, 0.0)))
    yidx = jnp.where(cnt == 1.0, ssum, tie_lo)
    yidx_ref[s] = yidx.astype(jnp.int32)
    y2xu_ref[s] = jnp.sqrt(col_min)


def _tc_stage(x, yt, x_normals):
    B = x.shape[0]
    return pl.pallas_call(
        _nn_body,
        grid=(B // _BPP,),
        in_specs=[
            pl.BlockSpec((_BPP, _P1, 3), lambda b: (b, 0, 0)),
            pl.BlockSpec((_BPP, 3, _P2), lambda b: (b, 0, 0)),
            pl.BlockSpec((_BPP, _P1, 3), lambda b: (b, 0, 0)),
        ],
        out_specs=[
            pl.BlockSpec((_BPP, 1, _P2), lambda b: (b, 0, 0)),
            pl.BlockSpec((_BPP, _P1, 1), lambda b: (b, 0, 0)),
            pl.BlockSpec((_BPP, 1, _P2), lambda b: (b, 0, 0)),
        ],
        out_shape=[
            jax.ShapeDtypeStruct((B, 1, _P2), jnp.float32),
            jax.ShapeDtypeStruct((B, _P1, 1), jnp.float32),
            jax.ShapeDtypeStruct((B, 1, _P2), jnp.int32),
        ],
    )(x, yt, x_normals)


_NGRP = (_P2 + 15) // 16


def _sc_sign_stage(tab, ytf, u, idx):
    info = plsc.get_sparse_core_info()
    nc = info.num_cores
    mesh = plsc.VectorSubcoreMesh(core_axis_name="c", subcore_axis_name="s")

    @functools.partial(
        pl.kernel,
        mesh=mesh,
        compiler_params=pltpu.CompilerParams(needs_layout_passes=False),
        out_type=jax.ShapeDtypeStruct((32 * _P2,), jnp.float32),
        scratch_types=[
            pltpu.VMEM((784 * 8,), jnp.float32),
            pltpu.VMEM((_P2,), jnp.float32),
            pltpu.VMEM((_P2,), jnp.float32),
            pltpu.VMEM((_P2,), jnp.float32),
            pltpu.VMEM((_P2,), jnp.float32),
            pltpu.VMEM((_P2,), jnp.int32),
            pltpu.VMEM((_P2,), jnp.float32),
        ],
    )
    def k(tab_hbm, ytf_hbm, u_hbm, idx_hbm, out_hbm,
          tab_v, y0_v, y1_v, y2_v, u_v, idx_v, out_v):
        wid = lax.axis_index("s") * nc + lax.axis_index("c")
        pltpu.sync_copy(tab_hbm.at[wid], tab_v)
        pltpu.sync_copy(ytf_hbm.at[pl.ds(wid * 3 * _P2 + 0 * _P2, _P2)], y0_v)
        pltpu.sync_copy(ytf_hbm.at[pl.ds(wid * 3 * _P2 + 1 * _P2, _P2)], y1_v)
        pltpu.sync_copy(ytf_hbm.at[pl.ds(wid * 3 * _P2 + 2 * _P2, _P2)], y2_v)
        pltpu.sync_copy(u_hbm.at[pl.ds(wid * _P2, _P2)], u_v)
        pltpu.sync_copy(idx_hbm.at[pl.ds(wid * _P2, _P2)], idx_v)

        def body(i, _):
            st = lax.min(i * 16, _P2 - 16)
            iv = idx_v[pl.ds(st, 16)] * 8
            g = [plsc.load_gather(tab_v, [iv + c]) for c in range(6)]
            y0 = y0_v[pl.ds(st, 16)]
            y1 = y1_v[pl.ds(st, 16)]
            y2 = y2_v[pl.ds(st, 16)]
            dot = g[0] * (y0 - g[3]) + g[1] * (y1 - g[4]) + g[2] * (y2 - g[5])
            out_v[pl.ds(st, 16)] = u_v[pl.ds(st, 16)] * jnp.sign(dot)
            return _

        lax.fori_loop(0, _NGRP, body, None)
        pltpu.sync_copy(out_v, out_hbm.at[pl.ds(wid * _P2, _P2)])

    return k(tab, ytf, u, idx)


@functools.partial(jax.jit, static_argnames=())
def kernel(x, y, x_normals):
    B = x.shape[0]
    yt = jnp.transpose(y, (0, 2, 1))

    y2xu, x2y_s, yidx = _tc_stage(x, yt, x_normals)

    tab = jnp.concatenate(
        [x_normals, x, jnp.zeros((B, _P1, 2), jnp.float32)], axis=2
    )
    tab = jnp.pad(tab, ((0, 0), (0, 784 - _P1), (0, 0)))

    y2x_signed = _sc_sign_stage(
        tab.reshape(B, 784 * 8),
        yt.reshape(B * 3 * _P2),
        y2xu.reshape(B * _P2),
        yidx.reshape(B * _P2),
    ).reshape(B, _P2)

    return (
        y2x_signed,
        x2y_s[:, :, 0],
        yidx[:, 0, :],
    )
